# baseline (speedup 1.0000x reference)
import jax
import jax.numpy as jnp
from jax import lax
from jax.experimental import pallas as pl
from jax.experimental.pallas import tpu as pltpu
from jax.experimental.pallas import tpu_sc as plsc

EMBED_DIM = 16
BATCH = 16384
NUM_CORES = 2
NUM_SUBCORES = 16
NUM_WORKERS = NUM_CORES * NUM_SUBCORES
BPW = BATCH // NUM_WORKERS
GROUPS = BPW // 16
IDX_TILES = BPW // 128


def _sc_body(x_hbm, table_hbm, w_hbm, b_hbm, out_hbm,
             idx_v, idx_s, rows_v, w_v, b_v, out_v, sem):
    wid = lax.axis_index("s") * NUM_CORES + lax.axis_index("c")

    pltpu.sync_copy(x_hbm.at[pl.ds(wid * IDX_TILES, IDX_TILES)], idx_v)
    for t in range(IDX_TILES):
        for u in range(8):
            seg = idx_v[t, pl.ds(u * 16, 16)]
            idx_s[t, pl.ds(u * 16, 16)] = seg >> 3

    copies = [
        pltpu.async_copy(
            table_hbm.at[idx_s.at[j]],
            rows_v.at[pl.ds(j * 128, 128)], sem)
        for j in range(IDX_TILES)
    ]
    for c in copies:
        c.wait()

    pltpu.sync_copy(w_hbm, w_v)
    pltpu.sync_copy(b_hbm, b_v)

    iota = lax.iota(jnp.int32, 16)
    w_splat = [w_v[d, :] for d in range(EMBED_DIM)]
    bias = b_v[...]

    def group(c, carry):
        t = c // 8
        u = c - t * 8
        sub = (idx_v[t, pl.ds(u * 16, 16)] & 7) * 16
        row_ids = c * 16 + iota
        acc = bias
        for d in range(EMBED_DIM):
            col = plsc.load_gather(rows_v, [row_ids, sub + d])
            acc = acc + col * w_splat[d]
        out_v[pl.ds(c * 16, 16)] = acc
        return carry

    lax.fori_loop(0, GROUPS, group, 0)
    pltpu.sync_copy(out_v, out_hbm.at[pl.ds(wid * BPW, BPW)])


@jax.jit
def kernel(x, table, W, b):
    w_bcast = jnp.broadcast_to(W.reshape(EMBED_DIM, 1), (EMBED_DIM, 16))
    b_splat = jnp.broadcast_to(b.reshape(()), (16,))
    x32 = x.astype(jnp.int32).reshape(BATCH // 128, 128)
    table128 = table.reshape(-1, 8 * EMBED_DIM)

    mesh = plsc.VectorSubcoreMesh(
        core_axis_name="c", subcore_axis_name="s",
        num_cores=NUM_CORES, num_subcores=NUM_SUBCORES)
    run = pl.kernel(
        _sc_body,
        mesh=mesh,
        out_type=jax.ShapeDtypeStruct((BATCH,), jnp.float32),
        scratch_types=[
            pltpu.VMEM((IDX_TILES, 128), jnp.int32),
            pltpu.VMEM((IDX_TILES, 128), jnp.int32),
            pltpu.VMEM((BPW, 8 * EMBED_DIM), jnp.float32),
            pltpu.VMEM((EMBED_DIM, 16), jnp.float32),
            pltpu.VMEM((16,), jnp.float32),
            pltpu.VMEM((BPW,), jnp.float32),
            pltpu.SemaphoreType.DMA,
        ],
        compiler_params=pltpu.CompilerParams(
            needs_layout_passes=False, use_tc_tiling_on_sc=True),
    )
    return run(x32, table128, w_bcast, b_splat)


# speedup vs baseline: 2.7801x; 2.7801x over previous
import jax
import jax.numpy as jnp
from jax import lax
from jax.experimental import pallas as pl
from jax.experimental.pallas import tpu as pltpu
from jax.experimental.pallas import tpu_sc as plsc

EMBED_DIM = 16
BATCH = 16384
VOCAB_ROWS = 1000000

S1_COLS = 5120
S1_ROWS = S1_COLS // 128
S1_GRID = 196
SCORE_ROWS = S1_GRID * S1_ROWS

NUM_CORES = 2
NUM_SUBCORES = 16
NUM_WORKERS = NUM_CORES * NUM_SUBCORES
BPW = BATCH // NUM_WORKERS
GROUPS = BPW // 16
IDX_TILES = BPW // 128


def _tc_dense_body(w_ref, t_ref, o_ref):
    r = lax.dot_general(w_ref[...], t_ref[...], (((1,), (0,)), ((), ())),
                        precision=lax.Precision.HIGHEST,
                        preferred_element_type=jnp.float32)
    o_ref[...] = r.reshape(o_ref.shape)


def _sc_gather_body(x_hbm, scores_hbm, b_hbm, out_hbm,
                    idx_v, idx_s, rows_v, b_v, out_v, sem):
    wid = lax.axis_index("s") * NUM_CORES + lax.axis_index("c")

    pltpu.sync_copy(x_hbm.at[pl.ds(wid * IDX_TILES, IDX_TILES)], idx_v)
    for t in range(IDX_TILES):
        for u in range(8):
            seg = idx_v[t, pl.ds(u * 16, 16)]
            idx_s[t, pl.ds(u * 16, 16)] = seg >> 7

    copies = [
        pltpu.async_copy(
            scores_hbm.at[idx_s.at[j]],
            rows_v.at[pl.ds(j * 128, 128)], sem)
        for j in range(IDX_TILES)
    ]
    for c in copies:
        c.wait()

    pltpu.sync_copy(b_hbm, b_v)
    iota = lax.iota(jnp.int32, 16)
    bias = b_v[...]

    def group(c, carry):
        t = c // 8
        u = c - t * 8
        sub = idx_v[t, pl.ds(u * 16, 16)] & 127
        row_ids = c * 16 + iota
        val = plsc.load_gather(rows_v, [row_ids, sub])
        out_v[pl.ds(c * 16, 16)] = val + bias
        return carry

    lax.fori_loop(0, GROUPS, group, 0)
    pltpu.sync_copy(out_v, out_hbm.at[pl.ds(wid * BPW, BPW)])


@jax.jit
def kernel(x, table, W, b):
    table_t = table.T
    b_splat = jnp.broadcast_to(b.reshape(()), (16,))
    x32 = x.astype(jnp.int32).reshape(BATCH // 128, 128)

    scores = pl.pallas_call(
        _tc_dense_body,
        grid=(S1_GRID,),
        in_specs=[
            pl.BlockSpec((1, EMBED_DIM), lambda k: (0, 0)),
            pl.BlockSpec((EMBED_DIM, S1_COLS), lambda k: (0, k)),
        ],
        out_specs=pl.BlockSpec((S1_ROWS, 128), lambda k: (k, 0)),
        out_shape=jax.ShapeDtypeStruct((SCORE_ROWS, 128), jnp.float32),
    )(W, table_t)

    mesh = plsc.VectorSubcoreMesh(
        core_axis_name="c", subcore_axis_name="s",
        num_cores=NUM_CORES, num_subcores=NUM_SUBCORES)
    run = pl.kernel(
        _sc_gather_body,
        mesh=mesh,
        out_type=jax.ShapeDtypeStruct((BATCH,), jnp.float32),
        scratch_types=[
            pltpu.VMEM((IDX_TILES, 128), jnp.int32),
            pltpu.VMEM((IDX_TILES, 128), jnp.int32),
            pltpu.VMEM((BPW, 128), jnp.float32),
            pltpu.VMEM((16,), jnp.float32),
            pltpu.VMEM((BPW,), jnp.float32),
            pltpu.SemaphoreType.DMA,
        ],
        compiler_params=pltpu.CompilerParams(
            needs_layout_passes=False, use_tc_tiling_on_sc=True),
    )
    return run(x32, scores, b_splat)


# speedup vs baseline: 3.4234x; 1.2314x over previous
import jax
import jax.numpy as jnp
from jax import lax
from jax.experimental import pallas as pl
from jax.experimental.pallas import tpu as pltpu
from jax.experimental.pallas import tpu_sc as plsc

EMBED_DIM = 16
BATCH = 16384
VOCAB_ROWS = 1000000

S1_COLS = 5120
S1_ROWS = S1_COLS // 128
S1_GRID = 196
SCORE_ROWS = S1_GRID * S1_ROWS

NUM_CORES = 2
NUM_SUBCORES = 16
NUM_WORKERS = NUM_CORES * NUM_SUBCORES
BPW = BATCH // NUM_WORKERS
GROUPS = BPW // 16
IDX_TILES = BPW // 128


def _tc_dense_body(w_ref, t_ref, o_ref):
    acc = t_ref[0, :].reshape(S1_ROWS, 128) * w_ref[0, 0]
    for d in range(1, EMBED_DIM):
        acc = acc + t_ref[d, :].reshape(S1_ROWS, 128) * w_ref[0, d]
    o_ref[...] = acc


def _sc_gather_body(x_hbm, scores_hbm, b_hbm, out_hbm,
                    idx_v, idx_s, rows_v, b_v, out_v, sem):
    wid = lax.axis_index("s") * NUM_CORES + lax.axis_index("c")

    pltpu.sync_copy(x_hbm.at[pl.ds(wid * IDX_TILES, IDX_TILES)], idx_v)
    for t in range(IDX_TILES):
        for u in range(8):
            seg = idx_v[t, pl.ds(u * 16, 16)]
            idx_s[t, pl.ds(u * 16, 16)] = seg >> 7

    copies = [
        pltpu.async_copy(
            scores_hbm.at[idx_s.at[j]],
            rows_v.at[pl.ds(j * 128, 128)], sem)
        for j in range(IDX_TILES)
    ]
    for c in copies:
        c.wait()

    pltpu.sync_copy(b_hbm, b_v)
    iota = lax.iota(jnp.int32, 16)
    bias = b_v[...]

    def group(c, carry):
        t = c // 8
        u = c - t * 8
        sub = idx_v[t, pl.ds(u * 16, 16)] & 127
        row_ids = c * 16 + iota
        val = plsc.load_gather(rows_v, [row_ids, sub])
        out_v[pl.ds(c * 16, 16)] = val + bias
        return carry

    lax.fori_loop(0, GROUPS, group, 0)
    pltpu.sync_copy(out_v, out_hbm.at[pl.ds(wid * BPW, BPW)])


@jax.jit
def kernel(x, table, W, b):
    table_t = table.T
    b_splat = jnp.broadcast_to(b.reshape(()), (16,))
    x32 = x.astype(jnp.int32).reshape(BATCH // 128, 128)

    scores = pl.pallas_call(
        _tc_dense_body,
        grid=(S1_GRID,),
        in_specs=[
            pl.BlockSpec(memory_space=pltpu.SMEM),
            pl.BlockSpec((EMBED_DIM, S1_COLS), lambda k: (0, k)),
        ],
        out_specs=pl.BlockSpec((S1_ROWS, 128), lambda k: (k, 0)),
        out_shape=jax.ShapeDtypeStruct((SCORE_ROWS, 128), jnp.float32),
    )(W, table_t)

    mesh = plsc.VectorSubcoreMesh(
        core_axis_name="c", subcore_axis_name="s",
        num_cores=NUM_CORES, num_subcores=NUM_SUBCORES)
    run = pl.kernel(
        _sc_gather_body,
        mesh=mesh,
        out_type=jax.ShapeDtypeStruct((BATCH,), jnp.float32),
        scratch_types=[
            pltpu.VMEM((IDX_TILES, 128), jnp.int32),
            pltpu.VMEM((IDX_TILES, 128), jnp.int32),
            pltpu.VMEM((BPW, 128), jnp.float32),
            pltpu.VMEM((16,), jnp.float32),
            pltpu.VMEM((BPW,), jnp.float32),
            pltpu.SemaphoreType.DMA,
        ],
        compiler_params=pltpu.CompilerParams(
            needs_layout_passes=False, use_tc_tiling_on_sc=True),
    )
    return run(x32, scores, b_splat)


# speedup vs baseline: 8.6975x; 2.5406x over previous
import jax
import jax.numpy as jnp
from jax import lax
from jax.experimental import pallas as pl
from jax.experimental.pallas import tpu as pltpu
from jax.experimental.pallas import tpu_sc as plsc

EMBED_DIM = 16
BATCH = 16384
VOCAB_ROWS = 1000000

S1_COLS = 40960
S1_ROWS = S1_COLS // 128
S1_GRID = 25
SCORE_ROWS = S1_GRID * S1_ROWS

NUM_CORES = 2
NUM_SUBCORES = 16
NUM_WORKERS = NUM_CORES * NUM_SUBCORES
BPW = BATCH // NUM_WORKERS
GROUPS = BPW // 16
IDX_TILES = BPW // 128


def _tc_dense_body(w_ref, t_ref, o_ref):
    acc = t_ref[0, :].reshape(S1_ROWS, 128) * w_ref[0, 0]
    for d in range(1, EMBED_DIM):
        acc = acc + t_ref[d, :].reshape(S1_ROWS, 128) * w_ref[0, d]
    o_ref[...] = acc


def _sc_gather_body(x_hbm, scores_hbm, b_hbm, out_hbm,
                    idx_v, idx_s, rows_v, b_v, out_v, sem):
    wid = lax.axis_index("s") * NUM_CORES + lax.axis_index("c")

    pltpu.sync_copy(x_hbm.at[pl.ds(wid * IDX_TILES, IDX_TILES)], idx_v)
    for t in range(IDX_TILES):
        for u in range(8):
            seg = idx_v[t, pl.ds(u * 16, 16)]
            idx_s[t, pl.ds(u * 16, 16)] = seg >> 7

    copies = [
        pltpu.async_copy(
            scores_hbm.at[idx_s.at[j]],
            rows_v.at[pl.ds(j * 128, 128)], sem)
        for j in range(IDX_TILES)
    ]
    for c in copies:
        c.wait()

    pltpu.sync_copy(b_hbm, b_v)
    iota = lax.iota(jnp.int32, 16)
    bias = b_v[...]

    def group(c, carry):
        t = c // 8
        u = c - t * 8
        sub = idx_v[t, pl.ds(u * 16, 16)] & 127
        row_ids = c * 16 + iota
        val = plsc.load_gather(rows_v, [row_ids, sub])
        out_v[pl.ds(c * 16, 16)] = val + bias
        return carry

    lax.fori_loop(0, GROUPS, group, 0)
    pltpu.sync_copy(out_v, out_hbm.at[pl.ds(wid * BPW, BPW)])


@jax.jit
def kernel(x, table, W, b):
    table_t = table.T
    b_splat = jnp.broadcast_to(b.reshape(()), (16,))
    x32 = x.astype(jnp.int32).reshape(BATCH // 128, 128)

    scores = pl.pallas_call(
        _tc_dense_body,
        grid=(S1_GRID,),
        in_specs=[
            pl.BlockSpec(memory_space=pltpu.SMEM),
            pl.BlockSpec((EMBED_DIM, S1_COLS), lambda k: (0, k)),
        ],
        out_specs=pl.BlockSpec((S1_ROWS, 128), lambda k: (k, 0)),
        out_shape=jax.ShapeDtypeStruct((SCORE_ROWS, 128), jnp.float32),
    )(W, table_t)

    mesh = plsc.VectorSubcoreMesh(
        core_axis_name="c", subcore_axis_name="s",
        num_cores=NUM_CORES, num_subcores=NUM_SUBCORES)
    run = pl.kernel(
        _sc_gather_body,
        mesh=mesh,
        out_type=jax.ShapeDtypeStruct((BATCH,), jnp.float32),
        scratch_types=[
            pltpu.VMEM((IDX_TILES, 128), jnp.int32),
            pltpu.VMEM((IDX_TILES, 128), jnp.int32),
            pltpu.VMEM((BPW, 128), jnp.float32),
            pltpu.VMEM((16,), jnp.float32),
            pltpu.VMEM((BPW,), jnp.float32),
            pltpu.SemaphoreType.DMA,
        ],
        compiler_params=pltpu.CompilerParams(
            needs_layout_passes=False, use_tc_tiling_on_sc=True),
    )
    return run(x32, scores, b_splat)


# speedup vs baseline: 9.7235x; 1.1180x over previous
import jax
import jax.numpy as jnp
from jax import lax
from jax.experimental import pallas as pl
from jax.experimental.pallas import tpu as pltpu
from jax.experimental.pallas import tpu_sc as plsc

EMBED_DIM = 16
BATCH = 16384
VOCAB_ROWS = 1000000

S1_COLS = 81920
S1_ROWS = S1_COLS // 128
S1_GRID = 13
SCORE_ROWS = S1_GRID * S1_ROWS

NUM_CORES = 2
NUM_SUBCORES = 16
NUM_WORKERS = NUM_CORES * NUM_SUBCORES
BPW = BATCH // NUM_WORKERS
GROUPS = BPW // 16
IDX_TILES = BPW // 128


def _tc_dense_body(w_ref, t_ref, o_ref):
    acc = t_ref[0, :].reshape(S1_ROWS, 128) * w_ref[0, 0]
    for d in range(1, EMBED_DIM):
        acc = acc + t_ref[d, :].reshape(S1_ROWS, 128) * w_ref[0, d]
    o_ref[...] = acc


def _sc_gather_body(x_hbm, scores_hbm, b_hbm, out_hbm,
                    idx_v, idx_s, rows_v, b_v, out_v, sem):
    wid = lax.axis_index("s") * NUM_CORES + lax.axis_index("c")

    pltpu.sync_copy(x_hbm.at[pl.ds(wid * IDX_TILES, IDX_TILES)], idx_v)
    for t in range(IDX_TILES):
        for u in range(8):
            seg = idx_v[t, pl.ds(u * 16, 16)]
            idx_s[t, pl.ds(u * 16, 16)] = seg >> 7

    copies = [
        pltpu.async_copy(
            scores_hbm.at[idx_s.at[j]],
            rows_v.at[pl.ds(j * 128, 128)], sem)
        for j in range(IDX_TILES)
    ]
    for c in copies:
        c.wait()

    pltpu.sync_copy(b_hbm, b_v)
    iota = lax.iota(jnp.int32, 16)
    bias = b_v[...]

    def group(c, carry):
        t = c // 8
        u = c - t * 8
        sub = idx_v[t, pl.ds(u * 16, 16)] & 127
        row_ids = c * 16 + iota
        val = plsc.load_gather(rows_v, [row_ids, sub])
        out_v[pl.ds(c * 16, 16)] = val + bias
        return carry

    lax.fori_loop(0, GROUPS, group, 0)
    pltpu.sync_copy(out_v, out_hbm.at[pl.ds(wid * BPW, BPW)])


@jax.jit
def kernel(x, table, W, b):
    table_t = table.T
    b_splat = jnp.broadcast_to(b.reshape(()), (16,))
    x32 = x.astype(jnp.int32).reshape(BATCH // 128, 128)

    scores = pl.pallas_call(
        _tc_dense_body,
        grid=(S1_GRID,),
        in_specs=[
            pl.BlockSpec(memory_space=pltpu.SMEM),
            pl.BlockSpec((EMBED_DIM, S1_COLS), lambda k: (0, k)),
        ],
        out_specs=pl.BlockSpec((S1_ROWS, 128), lambda k: (k, 0)),
        out_shape=jax.ShapeDtypeStruct((SCORE_ROWS, 128), jnp.float32),
    )(W, table_t)

    mesh = plsc.VectorSubcoreMesh(
        core_axis_name="c", subcore_axis_name="s",
        num_cores=NUM_CORES, num_subcores=NUM_SUBCORES)
    run = pl.kernel(
        _sc_gather_body,
        mesh=mesh,
        out_type=jax.ShapeDtypeStruct((BATCH,), jnp.float32),
        scratch_types=[
            pltpu.VMEM((IDX_TILES, 128), jnp.int32),
            pltpu.VMEM((IDX_TILES, 128), jnp.int32),
            pltpu.VMEM((BPW, 128), jnp.float32),
            pltpu.VMEM((16,), jnp.float32),
            pltpu.VMEM((BPW,), jnp.float32),
            pltpu.SemaphoreType.DMA,
        ],
        compiler_params=pltpu.CompilerParams(
            needs_layout_passes=False, use_tc_tiling_on_sc=True),
    )
    return run(x32, scores, b_splat)


# speedup vs baseline: 10.4445x; 1.0742x over previous
import jax
import jax.numpy as jnp
from jax import lax
from jax.experimental import pallas as pl
from jax.experimental.pallas import tpu as pltpu
from jax.experimental.pallas import tpu_sc as plsc

EMBED_DIM = 16
BATCH = 16384
VOCAB_ROWS = 1000000

S1_COLS = 102400
S1_ROWS = S1_COLS // 128
S1_GRID = 10
SCORE_ROWS = S1_GRID * S1_ROWS

NUM_CORES = 2
NUM_SUBCORES = 16
NUM_WORKERS = NUM_CORES * NUM_SUBCORES
BPW = BATCH // NUM_WORKERS
GROUPS = BPW // 16
IDX_TILES = BPW // 128


def _tc_dense_body(w_ref, b_ref, t_ref, o_ref):
    acc = jnp.full((S1_ROWS, 128), b_ref[0], jnp.float32)
    for d in range(EMBED_DIM):
        acc = acc + t_ref[d, :].reshape(S1_ROWS, 128) * w_ref[0, d]
    o_ref[...] = acc


def _sc_gather_body(x_hbm, scores_hbm, out_hbm,
                    idx_v, idx_s, rows_v, out_v, sem):
    wid = lax.axis_index("s") * NUM_CORES + lax.axis_index("c")

    pltpu.sync_copy(x_hbm.at[pl.ds(wid * IDX_TILES, IDX_TILES)], idx_v)
    for t in range(IDX_TILES):
        for u in range(8):
            seg = idx_v[t, pl.ds(u * 16, 16)]
            idx_s[t, pl.ds(u * 16, 16)] = seg >> 7

    copies = [
        pltpu.async_copy(
            scores_hbm.at[idx_s.at[j]],
            rows_v.at[pl.ds(j * 128, 128)], sem)
        for j in range(IDX_TILES)
    ]
    for c in copies:
        c.wait()

    iota = lax.iota(jnp.int32, 16)

    def group(c, carry):
        t = c // 8
        u = c - t * 8
        sub = idx_v[t, pl.ds(u * 16, 16)] & 127
        row_ids = c * 16 + iota
        out_v[pl.ds(c * 16, 16)] = plsc.load_gather(rows_v, [row_ids, sub])
        return carry

    lax.fori_loop(0, GROUPS, group, 0)
    pltpu.sync_copy(out_v, out_hbm.at[pl.ds(wid * BPW, BPW)])


@jax.jit
def kernel(x, table, W, b):
    table_t = table.T
    x32 = x.astype(jnp.int32).reshape(BATCH // 128, 128)

    scores = pl.pallas_call(
        _tc_dense_body,
        grid=(S1_GRID,),
        in_specs=[
            pl.BlockSpec(memory_space=pltpu.SMEM),
            pl.BlockSpec(memory_space=pltpu.SMEM),
            pl.BlockSpec((EMBED_DIM, S1_COLS), lambda k: (0, k)),
        ],
        out_specs=pl.BlockSpec((S1_ROWS, 128), lambda k: (k, 0)),
        out_shape=jax.ShapeDtypeStruct((SCORE_ROWS, 128), jnp.float32),
    )(W, b, table_t)

    mesh = plsc.VectorSubcoreMesh(
        core_axis_name="c", subcore_axis_name="s",
        num_cores=NUM_CORES, num_subcores=NUM_SUBCORES)
    run = pl.kernel(
        _sc_gather_body,
        mesh=mesh,
        out_type=jax.ShapeDtypeStruct((BATCH,), jnp.float32),
        scratch_types=[
            pltpu.VMEM((IDX_TILES, 128), jnp.int32),
            pltpu.VMEM((IDX_TILES, 128), jnp.int32),
            pltpu.VMEM((BPW, 128), jnp.float32),
            pltpu.VMEM((BPW,), jnp.float32),
            pltpu.SemaphoreType.DMA,
        ],
        compiler_params=pltpu.CompilerParams(
            needs_layout_passes=False, use_tc_tiling_on_sc=True),
    )
    return run(x32, scores)
